# Initial kernel scaffold; baseline (speedup 1.0000x reference)
import jax
import jax.numpy as jnp
from jax.experimental import pallas as pl


def kernel(x, edge_index, W1, b1, W2, b2, W3, b3, W4, b4, Wc, bc):
    raise NotImplementedError("write your pallas kernel here")


# speedup vs baseline: 18.2943x; 18.2943x over previous
import functools

import jax
import jax.numpy as jnp
from jax import lax
from jax.experimental import pallas as pl
from jax.experimental.pallas import tpu as pltpu
from jax.experimental.pallas import tpu_sc as plsc

NC = 2
NS = 16
NW = NC * NS

_R = 2000


def _make_deg_kernel(n, e):
    t = 2000
    epw = e // NW
    nt = epw // t

    mesh = plsc.VectorSubcoreMesh(core_axis_name="c", subcore_axis_name="s")

    @functools.partial(
        pl.kernel,
        out_type=jax.ShapeDtypeStruct((NC, n), jnp.float32),
        mesh=mesh,
        compiler_params=pltpu.CompilerParams(use_tc_tiling_on_sc=False),
        scratch_types=[
            pltpu.VMEM((t,), jnp.int32),
            pltpu.VMEM((t,), jnp.float32),
            pltpu.VMEM_SHARED((n,), jnp.float32),
        ],
    )
    def deg_kernel(dst_hbm, out_hbm, dst_v, one_v, acc_sh):
        c = lax.axis_index("c")
        s = lax.axis_index("s")

        def fill(val, i, _):
            one_v[pl.ds(i * 16, 16)] = jnp.full((16,), val, jnp.float32)
            return 0

        @pl.when(s == 0)
        def _():
            lax.fori_loop(0, t // 16, functools.partial(fill, 0.0), 0)
            for k in range(n // t):
                pltpu.sync_copy(one_v, acc_sh.at[pl.ds(k * t, t)])

        lax.fori_loop(0, t // 16, functools.partial(fill, 1.0), 0)
        plsc.subcore_barrier()

        base = (c * NS + s) * epw

        def body(i, _):
            pltpu.sync_copy(dst_hbm.at[pl.ds(base + i * t, t)], dst_v)
            pltpu.sync_copy(one_v, acc_sh.at[dst_v], add=True)
            return 0

        lax.fori_loop(0, nt, body, 0)
        plsc.subcore_barrier()

        @pl.when(s == 0)
        def _():
            pltpu.sync_copy(acc_sh, out_hbm.at[c])

    return deg_kernel


def _make_agg_kernel(n, e):
    t = 1000
    epw = e // NW
    nt = epw // t
    n_pad = ((n + 8 * NS - 1) // (8 * NS)) * (8 * NS)
    rpt = n_pad // NS

    mesh = plsc.VectorSubcoreMesh(core_axis_name="c", subcore_axis_name="s")

    @functools.partial(
        pl.kernel,
        out_type=jax.ShapeDtypeStruct((NC, n_pad, 16), jnp.float32),
        mesh=mesh,
        compiler_params=pltpu.CompilerParams(use_tc_tiling_on_sc=False),
        scratch_types=[
            pltpu.VMEM((t,), jnp.int32),
            pltpu.VMEM((t,), jnp.int32),
            pltpu.VMEM((t, 16), jnp.float32),
            pltpu.VMEM_SHARED((n_pad, 16), jnp.float32),
            pltpu.SemaphoreType.DMA,
        ],
    )
    def agg_kernel(src_hbm, dst_hbm, tab_hbm, out_hbm,
                   src_v, dst_v, rows_v, acc_sh, sem):
        c = lax.axis_index("c")
        s = lax.axis_index("s")

        def zfill(i, _):
            rows_v[i] = jnp.zeros((16,), jnp.float32)
            return 0

        lax.fori_loop(0, t, zfill, 0)
        row0 = s * rpt
        nfull, rem = divmod(rpt, t)
        for k in range(nfull):
            pltpu.sync_copy(rows_v, acc_sh.at[pl.ds(row0 + k * t, t)])
        if rem:
            pltpu.sync_copy(rows_v.at[pl.ds(0, rem)],
                            acc_sh.at[pl.ds(row0 + nfull * t, rem)])
        plsc.subcore_barrier()

        base = (c * NS + s) * epw

        def body(i, _):
            off = base + i * t
            pltpu.sync_copy(src_hbm.at[pl.ds(off, t)], src_v)
            pltpu.sync_copy(dst_hbm.at[pl.ds(off, t)], dst_v)
            pltpu.async_copy(tab_hbm.at[src_v], rows_v, sem).wait()
            pltpu.sync_copy(rows_v, acc_sh.at[dst_v], add=True)
            return 0

        lax.fori_loop(0, nt, body, 0)
        plsc.subcore_barrier()
        pltpu.sync_copy(acc_sh.at[pl.ds(row0, rpt)],
                        out_hbm.at[c, pl.ds(row0, rpt)])

    return agg_kernel


def _tc_call(body, n, out_shapes, in_specs, out_specs):
    grid = (n // _R,)
    return pl.pallas_call(
        body,
        grid=grid,
        out_shape=out_shapes,
        in_specs=in_specs,
        out_specs=out_specs,
    )


def _full(shape):
    return pl.BlockSpec(shape, lambda i: tuple(0 for _ in shape))


def _rows(shape, axis):
    def imap(i, axis=axis):
        return tuple(i if a == axis else 0 for a in range(len(shape)))
    return pl.BlockSpec(shape, imap)


def _tcA(deg2, x):
    n = x.shape[0]

    def body(deg_ref, x_ref, dis_ref, g0_ref):
        deg = deg_ref[0] + deg_ref[1] + 1.0
        dis = lax.rsqrt(deg)
        dis_ref[...] = dis
        xp = jnp.concatenate(
            [x_ref[...], jnp.zeros((_R, 14), jnp.float32)], axis=1)
        g = dis * xp
        for c in range(3):
            g0_ref[c] = g[:, c * 16:(c + 1) * 16]

    return _tc_call(
        body, n,
        (jax.ShapeDtypeStruct((n, 1), jnp.float32),
         jax.ShapeDtypeStruct((3, n, 16), jnp.float32)),
        [_rows((2, _R, 1), 1), _rows((_R, 34), 0)],
        (_rows((_R, 1), 0), _rows((3, _R, 16), 1)),
    )(deg2, x)


def _tcB1(agg0, g0, dis, W1, b1, W2):
    n = dis.shape[0]

    def body(a_ref, g_ref, d_ref, w1_ref, b1_ref, w2_ref, g1_ref):
        a = a_ref[0] + a_ref[1] + g_ref[...]
        u = jnp.concatenate([a[0], a[1], a[2]], axis=1)
        d = d_ref[...]
        pre = (d * u)[:, :34]
        h1 = jnp.tanh(
            jnp.dot(pre, w1_ref[...], preferred_element_type=jnp.float32)
            + b1_ref[...])
        m2 = jnp.dot(h1, w2_ref[...], preferred_element_type=jnp.float32)
        g1 = d * m2
        g1_ref[0] = g1[:, :16]
        g1_ref[1] = g1[:, 16:]

    return _tc_call(
        body, n,
        jax.ShapeDtypeStruct((2, n, 16), jnp.float32),
        [_rows((2, 3, _R, 16), 2), _rows((3, _R, 16), 1), _rows((_R, 1), 0),
         _full((34, 64)), _full((1, 64)), _full((64, 32))],
        _rows((2, _R, 16), 1),
    )(agg0, g0, dis, W1, b1, W2)


def _tcB2(agg1, g1, dis, b2, W3):
    n = dis.shape[0]

    def body(a_ref, g_ref, d_ref, b2_ref, w3_ref, g2_ref):
        a = a_ref[0] + a_ref[1] + g_ref[...]
        u = jnp.concatenate([a[0], a[1]], axis=1)
        d = d_ref[...]
        h2 = jnp.tanh(d * u + b2_ref[...])
        m3 = jnp.dot(h2, w3_ref[...], preferred_element_type=jnp.float32)
        g2_ref[0] = d * m3

    return _tc_call(
        body, n,
        jax.ShapeDtypeStruct((1, n, 16), jnp.float32),
        [_rows((2, 2, _R, 16), 2), _rows((2, _R, 16), 1), _rows((_R, 1), 0),
         _full((1, 32)), _full((32, 16))],
        _rows((1, _R, 16), 1),
    )(agg1, g1, dis, b2, W3)


def _tcB3(agg2, g2, dis, b3, W4):
    n = dis.shape[0]

    def body(a_ref, g_ref, d_ref, b3_ref, w4_ref, g3_ref):
        u = a_ref[0, 0] + a_ref[1, 0] + g_ref[0]
        d = d_ref[...]
        h3 = jnp.tanh(d * u + b3_ref[...])
        m4 = jnp.dot(h3, w4_ref[...], preferred_element_type=jnp.float32)
        g3 = d * m4
        g3_ref[0] = jnp.concatenate(
            [g3, jnp.zeros((_R, 8), jnp.float32)], axis=1)

    return _tc_call(
        body, n,
        jax.ShapeDtypeStruct((1, n, 16), jnp.float32),
        [_rows((2, 1, _R, 16), 2), _rows((1, _R, 16), 1), _rows((_R, 1), 0),
         _full((1, 16)), _full((16, 8))],
        _rows((1, _R, 16), 1),
    )(agg2, g2, dis, b3, W4)


def _tcC(agg3, g3, dis, b4, Wc, bc):
    n = dis.shape[0]

    def body(a_ref, g_ref, d_ref, b4_ref, wc_ref, bc_ref, out_ref, h4_ref):
        u = (a_ref[0, 0] + a_ref[1, 0] + g_ref[0])[:, :8]
        d = d_ref[...]
        h4 = jnp.tanh(d * u + b4_ref[...])
        h4_ref[...] = h4
        out_ref[...] = (
            jnp.dot(h4, wc_ref[...], preferred_element_type=jnp.float32)
            + bc_ref[...])

    return _tc_call(
        body, n,
        (jax.ShapeDtypeStruct((n, 2), jnp.float32),
         jax.ShapeDtypeStruct((n, 8), jnp.float32)),
        [_rows((2, 1, _R, 16), 2), _rows((1, _R, 16), 1), _rows((_R, 1), 0),
         _full((1, 8)), _full((8, 2)), _full((1, 2))],
        (_rows((_R, 2), 0), _rows((_R, 8), 0)),
    )(agg3, g3, dis, b4, Wc, bc)


def kernel(x, edge_index, W1, b1, W2, b2, W3, b3, W4, b4, Wc, bc):
    n = x.shape[0]
    e = edge_index.shape[1]
    src = edge_index[0]
    dst = edge_index[1]

    deg_k = _make_deg_kernel(n, e)
    agg_k = _make_agg_kernel(n, e)

    deg2 = deg_k(dst).reshape(NC, n, 1)
    dis, g0 = _tcA(deg2, x)

    agg0 = jnp.stack([agg_k(src, dst, g0[c]) for c in range(3)], axis=1)
    g1 = _tcB1(agg0, g0, dis, W1, b1.reshape(1, -1), W2)

    agg1 = jnp.stack([agg_k(src, dst, g1[c]) for c in range(2)], axis=1)
    g2 = _tcB2(agg1, g1, dis, b2.reshape(1, -1), W3)

    agg2 = agg_k(src, dst, g2[0])[:, None]
    g3 = _tcB3(agg2, g2, dis, b3.reshape(1, -1), W4)

    agg3 = agg_k(src, dst, g3[0])[:, None]
    out, h4 = _tcC(agg3, g3, dis, b4.reshape(1, -1), Wc, bc.reshape(1, -1))
    return (out, h4)


# speedup vs baseline: 21.2171x; 1.1598x over previous
import functools

import jax
import jax.numpy as jnp
from jax import lax
from jax.experimental import pallas as pl
from jax.experimental.pallas import tpu as pltpu
from jax.experimental.pallas import tpu_sc as plsc

NC = 2
NS = 16
NW = NC * NS

_R = 2000


def _make_deg_kernel(n, e):
    t = 2000
    epw = e // NW
    nt = epw // t

    mesh = plsc.VectorSubcoreMesh(core_axis_name="c", subcore_axis_name="s")

    @functools.partial(
        pl.kernel,
        out_type=jax.ShapeDtypeStruct((NC, n), jnp.float32),
        mesh=mesh,
        compiler_params=pltpu.CompilerParams(use_tc_tiling_on_sc=False),
        scratch_types=[
            pltpu.VMEM((t,), jnp.int32),
            pltpu.VMEM((t,), jnp.float32),
            pltpu.VMEM_SHARED((n,), jnp.float32),
        ],
    )
    def deg_kernel(edge_hbm, out_hbm, dst_v, one_v, acc_sh):
        c = lax.axis_index("c")
        s = lax.axis_index("s")

        def fill(val, i, _):
            one_v[pl.ds(i * 16, 16)] = jnp.full((16,), val, jnp.float32)
            return 0

        @pl.when(s == 0)
        def _():
            lax.fori_loop(0, t // 16, functools.partial(fill, 0.0), 0)
            for k in range(n // t):
                pltpu.sync_copy(one_v, acc_sh.at[pl.ds(k * t, t)])

        lax.fori_loop(0, t // 16, functools.partial(fill, 1.0), 0)
        plsc.subcore_barrier()

        base = (c * NS + s) * epw

        def body(i, _):
            pltpu.sync_copy(edge_hbm.at[1, pl.ds(base + i * t, t)], dst_v)
            pltpu.sync_copy(one_v, acc_sh.at[dst_v], add=True)
            return 0

        lax.fori_loop(0, nt, body, 0)
        plsc.subcore_barrier()

        @pl.when(s == 0)
        def _():
            pltpu.sync_copy(acc_sh, out_hbm.at[c])

    return deg_kernel


def _make_agg_kernel(n, e, chunk):
    t = 1000
    epw = e // NW
    nt = epw // t
    n_pad = ((n + 8 * NS - 1) // (8 * NS)) * (8 * NS)
    rpt = n_pad // NS

    mesh = plsc.VectorSubcoreMesh(core_axis_name="c", subcore_axis_name="s")

    @functools.partial(
        pl.kernel,
        out_type=jax.ShapeDtypeStruct((NC, n_pad, 16), jnp.float32),
        mesh=mesh,
        compiler_params=pltpu.CompilerParams(use_tc_tiling_on_sc=False),
        scratch_types=[
            pltpu.VMEM((t,), jnp.int32),
            pltpu.VMEM((t,), jnp.int32),
            pltpu.VMEM((t, 16), jnp.float32),
            pltpu.VMEM_SHARED((n_pad, 16), jnp.float32),
            pltpu.SemaphoreType.DMA,
        ],
    )
    def agg_kernel(edge_hbm, tab3_hbm, out_hbm,
                   src_v, dst_v, rows_v, acc_sh, sem):
        c = lax.axis_index("c")
        s = lax.axis_index("s")
        tab_hbm = tab3_hbm.at[chunk]

        def zfill(i, _):
            rows_v[i] = jnp.zeros((16,), jnp.float32)
            return 0

        lax.fori_loop(0, t, zfill, 0)
        row0 = s * rpt
        nfull, rem = divmod(rpt, t)
        for k in range(nfull):
            pltpu.sync_copy(rows_v, acc_sh.at[pl.ds(row0 + k * t, t)])
        if rem:
            pltpu.sync_copy(rows_v.at[pl.ds(0, rem)],
                            acc_sh.at[pl.ds(row0 + nfull * t, rem)])
        plsc.subcore_barrier()

        base = (c * NS + s) * epw

        def body(i, _):
            off = base + i * t
            pltpu.sync_copy(edge_hbm.at[0, pl.ds(off, t)], src_v)
            pltpu.sync_copy(edge_hbm.at[1, pl.ds(off, t)], dst_v)
            pltpu.async_copy(tab_hbm.at[src_v], rows_v, sem).wait()
            pltpu.sync_copy(rows_v, acc_sh.at[dst_v], add=True)
            return 0

        lax.fori_loop(0, nt, body, 0)
        plsc.subcore_barrier()
        pltpu.sync_copy(acc_sh.at[pl.ds(row0, rpt)],
                        out_hbm.at[c, pl.ds(row0, rpt)])

    return agg_kernel


def _tc_call(body, n, out_shapes, in_specs, out_specs):
    grid = (n // _R,)
    return pl.pallas_call(
        body,
        grid=grid,
        out_shape=out_shapes,
        in_specs=in_specs,
        out_specs=out_specs,
    )


def _full(shape):
    return pl.BlockSpec(shape, lambda i: tuple(0 for _ in shape))


def _rows(shape, axis):
    def imap(i, axis=axis):
        return tuple(i if a == axis else 0 for a in range(len(shape)))
    return pl.BlockSpec(shape, imap)


def _tcA(deg2, x):
    n = x.shape[0]

    def body(deg_ref, x_ref, dis_ref, g0_ref):
        deg = deg_ref[0] + deg_ref[1] + 1.0
        dis = lax.rsqrt(deg)
        dis_ref[...] = dis
        xp = jnp.concatenate(
            [x_ref[...], jnp.zeros((_R, 14), jnp.float32)], axis=1)
        g = dis * xp
        for c in range(3):
            g0_ref[c] = g[:, c * 16:(c + 1) * 16]

    return _tc_call(
        body, n,
        (jax.ShapeDtypeStruct((n, 1), jnp.float32),
         jax.ShapeDtypeStruct((3, n, 16), jnp.float32)),
        [_rows((2, _R, 1), 1), _rows((_R, 34), 0)],
        (_rows((_R, 1), 0), _rows((3, _R, 16), 1)),
    )(deg2, x)


def _agg_specs(n_arr, count):
    return [_rows((2, _R, 16), 1) for _ in range(count)]


def _tcB1(a0, a1, a2, g0, dis, W1, b1, W2):
    n = dis.shape[0]

    def body(a0_ref, a1_ref, a2_ref, g_ref, d_ref,
             w1_ref, b1_ref, w2_ref, g1_ref):
        arefs = (a0_ref, a1_ref, a2_ref)
        cols = [arefs[c][0] + arefs[c][1] + g_ref[c] for c in range(3)]
        u = jnp.concatenate(cols, axis=1)
        d = d_ref[...]
        pre = (d * u)[:, :34]
        h1 = jnp.tanh(
            jnp.dot(pre, w1_ref[...], preferred_element_type=jnp.float32)
            + b1_ref[...])
        m2 = jnp.dot(h1, w2_ref[...], preferred_element_type=jnp.float32)
        g1 = d * m2
        g1_ref[0] = g1[:, :16]
        g1_ref[1] = g1[:, 16:]

    return _tc_call(
        body, n,
        jax.ShapeDtypeStruct((2, n, 16), jnp.float32),
        _agg_specs(n, 3) + [_rows((3, _R, 16), 1), _rows((_R, 1), 0),
                            _full((34, 64)), _full((1, 64)), _full((64, 32))],
        _rows((2, _R, 16), 1),
    )(a0, a1, a2, g0, dis, W1, b1, W2)


def _tcB2(a0, a1, g1, dis, b2, W3):
    n = dis.shape[0]

    def body(a0_ref, a1_ref, g_ref, d_ref, b2_ref, w3_ref, g2_ref):
        arefs = (a0_ref, a1_ref)
        cols = [arefs[c][0] + arefs[c][1] + g_ref[c] for c in range(2)]
        u = jnp.concatenate(cols, axis=1)
        d = d_ref[...]
        h2 = jnp.tanh(d * u + b2_ref[...])
        m3 = jnp.dot(h2, w3_ref[...], preferred_element_type=jnp.float32)
        g2_ref[0] = d * m3

    return _tc_call(
        body, n,
        jax.ShapeDtypeStruct((1, n, 16), jnp.float32),
        _agg_specs(n, 2) + [_rows((2, _R, 16), 1), _rows((_R, 1), 0),
                            _full((1, 32)), _full((32, 16))],
        _rows((1, _R, 16), 1),
    )(a0, a1, g1, dis, b2, W3)


def _tcB3(a0, g2, dis, b3, W4):
    n = dis.shape[0]

    def body(a0_ref, g_ref, d_ref, b3_ref, w4_ref, g3_ref):
        u = a0_ref[0] + a0_ref[1] + g_ref[0]
        d = d_ref[...]
        h3 = jnp.tanh(d * u + b3_ref[...])
        m4 = jnp.dot(h3, w4_ref[...], preferred_element_type=jnp.float32)
        g3 = d * m4
        g3_ref[0] = jnp.concatenate(
            [g3, jnp.zeros((_R, 8), jnp.float32)], axis=1)

    return _tc_call(
        body, n,
        jax.ShapeDtypeStruct((1, n, 16), jnp.float32),
        _agg_specs(n, 1) + [_rows((1, _R, 16), 1), _rows((_R, 1), 0),
                            _full((1, 16)), _full((16, 8))],
        _rows((1, _R, 16), 1),
    )(a0, g2, dis, b3, W4)


def _tcC(a0, g3, dis, b4, Wc, bc):
    n = dis.shape[0]

    def body(a0_ref, g_ref, d_ref, b4_ref, wc_ref, bc_ref, out_ref, h4_ref):
        u = (a0_ref[0] + a0_ref[1] + g_ref[0])[:, :8]
        d = d_ref[...]
        h4 = jnp.tanh(d * u + b4_ref[...])
        h4_ref[...] = h4
        out_ref[...] = (
            jnp.dot(h4, wc_ref[...], preferred_element_type=jnp.float32)
            + bc_ref[...])

    return _tc_call(
        body, n,
        (jax.ShapeDtypeStruct((n, 2), jnp.float32),
         jax.ShapeDtypeStruct((n, 8), jnp.float32)),
        _agg_specs(n, 1) + [_rows((1, _R, 16), 1), _rows((_R, 1), 0),
                            _full((1, 8)), _full((8, 2)), _full((1, 2))],
        (_rows((_R, 2), 0), _rows((_R, 8), 0)),
    )(a0, g3, dis, b4, Wc, bc)


def kernel(x, edge_index, W1, b1, W2, b2, W3, b3, W4, b4, Wc, bc):
    n = x.shape[0]
    e = edge_index.shape[1]

    deg_k = _make_deg_kernel(n, e)
    agg_ks = [_make_agg_kernel(n, e, c) for c in range(3)]

    deg2 = deg_k(edge_index).reshape(NC, n, 1)
    dis, g0 = _tcA(deg2, x)

    a00 = agg_ks[0](edge_index, g0)
    a01 = agg_ks[1](edge_index, g0)
    a02 = agg_ks[2](edge_index, g0)
    g1 = _tcB1(a00, a01, a02, g0, dis, W1, b1.reshape(1, -1), W2)

    a10 = agg_ks[0](edge_index, g1)
    a11 = agg_ks[1](edge_index, g1)
    g2 = _tcB2(a10, a11, g1, dis, b2.reshape(1, -1), W3)

    a20 = agg_ks[0](edge_index, g2)
    g3 = _tcB3(a20, g2, dis, b3.reshape(1, -1), W4)

    a30 = agg_ks[0](edge_index, g3)
    out, h4 = _tcC(a30, g3, dis, b4.reshape(1, -1), Wc, bc.reshape(1, -1))
    return (out, h4)


# speedup vs baseline: 25.1937x; 1.1874x over previous
import functools

import jax
import jax.numpy as jnp
from jax import lax
from jax.experimental import pallas as pl
from jax.experimental.pallas import tpu as pltpu
from jax.experimental.pallas import tpu_sc as plsc

NC = 2
NS = 16
NW = NC * NS

_R = 2000


def _make_deg_kernel(n, e):
    t = 2000
    epw = e // NW
    nt = epw // t

    mesh = plsc.VectorSubcoreMesh(core_axis_name="c", subcore_axis_name="s")

    @functools.partial(
        pl.kernel,
        out_type=jax.ShapeDtypeStruct((NC, n), jnp.float32),
        mesh=mesh,
        compiler_params=pltpu.CompilerParams(use_tc_tiling_on_sc=False),
        scratch_types=[
            pltpu.VMEM((t,), jnp.int32),
            pltpu.VMEM((t,), jnp.float32),
            pltpu.VMEM_SHARED((n,), jnp.float32),
        ],
    )
    def deg_kernel(edge_hbm, out_hbm, dst_v, one_v, acc_sh):
        c = lax.axis_index("c")
        s = lax.axis_index("s")

        def fill(val, i, _):
            one_v[pl.ds(i * 16, 16)] = jnp.full((16,), val, jnp.float32)
            return 0

        @pl.when(s == 0)
        def _():
            lax.fori_loop(0, t // 16, functools.partial(fill, 0.0), 0)
            for k in range(n // t):
                pltpu.sync_copy(one_v, acc_sh.at[pl.ds(k * t, t)])

        lax.fori_loop(0, t // 16, functools.partial(fill, 1.0), 0)
        plsc.subcore_barrier()

        base = (c * NS + s) * epw

        def body(i, _):
            pltpu.sync_copy(edge_hbm.at[1, pl.ds(base + i * t, t)], dst_v)
            pltpu.sync_copy(one_v, acc_sh.at[dst_v], add=True)
            return 0

        lax.fori_loop(0, nt, body, 0)
        plsc.subcore_barrier()

        @pl.when(s == 0)
        def _():
            pltpu.sync_copy(acc_sh, out_hbm.at[c])

    return deg_kernel


def _make_agg_kernel(n, e, chunk):
    t = 200
    k_slots = 5
    tb = t * k_slots
    epw = e // NW
    nb = epw // tb
    n_pad = ((n + 8 * NS - 1) // (8 * NS)) * (8 * NS)
    rpt = n_pad // NS

    mesh = plsc.VectorSubcoreMesh(core_axis_name="c", subcore_axis_name="s")

    @functools.partial(
        pl.kernel,
        out_type=jax.ShapeDtypeStruct((NC, n_pad, 16), jnp.float32),
        mesh=mesh,
        compiler_params=pltpu.CompilerParams(use_tc_tiling_on_sc=False),
        scratch_types=[
            pltpu.VMEM((k_slots, 2, t), jnp.int32),
            pltpu.VMEM((tb, 16), jnp.float32),
            pltpu.VMEM_SHARED((n_pad, 16), jnp.float32),
            pltpu.SemaphoreType.DMA((k_slots,)),
            pltpu.SemaphoreType.DMA((k_slots,)),
            pltpu.SemaphoreType.DMA((k_slots,)),
        ],
    )
    def agg_kernel(edge_hbm, tab3_hbm, out_hbm,
                   idx_v, rows_v, acc_sh, isem, gsem, ssem):
        c = lax.axis_index("c")
        s = lax.axis_index("s")
        tab_hbm = tab3_hbm.at[chunk]

        def zfill(i, _):
            rows_v[i] = jnp.zeros((16,), jnp.float32)
            return 0

        lax.fori_loop(0, tb, zfill, 0)
        row0 = s * rpt
        nfull, rem = divmod(rpt, tb)
        for k in range(nfull):
            pltpu.sync_copy(rows_v, acc_sh.at[pl.ds(row0 + k * tb, tb)])
        if rem:
            pltpu.sync_copy(rows_v.at[pl.ds(0, rem)],
                            acc_sh.at[pl.ds(row0 + nfull * tb, rem)])
        plsc.subcore_barrier()

        base = (c * NS + s) * epw

        def body(m, _):
            off0 = base + m * tb
            ic, gc, sc = [], [], []
            for b in range(k_slots):
                ic.append(pltpu.async_copy(
                    edge_hbm.at[:, pl.ds(off0 + b * t, t)],
                    idx_v.at[b], isem.at[b]))
            for b in range(k_slots):
                ic[b].wait()
                gc.append(pltpu.async_copy(
                    tab_hbm.at[idx_v.at[b, 0]],
                    rows_v.at[pl.ds(b * t, t)], gsem.at[b]))
            for b in range(k_slots):
                gc[b].wait()
                sc.append(pltpu.async_copy(
                    rows_v.at[pl.ds(b * t, t)],
                    acc_sh.at[idx_v.at[b, 1]], ssem.at[b], add=True))
            for b in range(k_slots):
                sc[b].wait()
            return 0

        lax.fori_loop(0, nb, body, 0)
        plsc.subcore_barrier()
        pltpu.sync_copy(acc_sh.at[pl.ds(row0, rpt)],
                        out_hbm.at[c, pl.ds(row0, rpt)])

    return agg_kernel


def _tc_call(body, n, out_shapes, in_specs, out_specs):
    grid = (n // _R,)
    return pl.pallas_call(
        body,
        grid=grid,
        out_shape=out_shapes,
        in_specs=in_specs,
        out_specs=out_specs,
    )


def _full(shape):
    return pl.BlockSpec(shape, lambda i: tuple(0 for _ in shape))


def _rows(shape, axis):
    def imap(i, axis=axis):
        return tuple(i if a == axis else 0 for a in range(len(shape)))
    return pl.BlockSpec(shape, imap)


def _tcA(deg2, x):
    n = x.shape[0]

    def body(deg_ref, x_ref, dis_ref, g0_ref):
        deg = deg_ref[0] + deg_ref[1] + 1.0
        dis = lax.rsqrt(deg)
        dis_ref[...] = dis
        xp = jnp.concatenate(
            [x_ref[...], jnp.zeros((_R, 14), jnp.float32)], axis=1)
        g = dis * xp
        for c in range(3):
            g0_ref[c] = g[:, c * 16:(c + 1) * 16]

    return _tc_call(
        body, n,
        (jax.ShapeDtypeStruct((n, 1), jnp.float32),
         jax.ShapeDtypeStruct((3, n, 16), jnp.float32)),
        [_rows((2, _R, 1), 1), _rows((_R, 34), 0)],
        (_rows((_R, 1), 0), _rows((3, _R, 16), 1)),
    )(deg2, x)


def _agg_specs(n_arr, count):
    return [_rows((2, _R, 16), 1) for _ in range(count)]


def _tcB1(a0, a1, a2, g0, dis, W1, b1, W2):
    n = dis.shape[0]

    def body(a0_ref, a1_ref, a2_ref, g_ref, d_ref,
             w1_ref, b1_ref, w2_ref, g1_ref):
        arefs = (a0_ref, a1_ref, a2_ref)
        cols = [arefs[c][0] + arefs[c][1] + g_ref[c] for c in range(3)]
        u = jnp.concatenate(cols, axis=1)
        d = d_ref[...]
        pre = (d * u)[:, :34]
        h1 = jnp.tanh(
            jnp.dot(pre, w1_ref[...], preferred_element_type=jnp.float32)
            + b1_ref[...])
        m2 = jnp.dot(h1, w2_ref[...], preferred_element_type=jnp.float32)
        g1 = d * m2
        g1_ref[0] = g1[:, :16]
        g1_ref[1] = g1[:, 16:]

    return _tc_call(
        body, n,
        jax.ShapeDtypeStruct((2, n, 16), jnp.float32),
        _agg_specs(n, 3) + [_rows((3, _R, 16), 1), _rows((_R, 1), 0),
                            _full((34, 64)), _full((1, 64)), _full((64, 32))],
        _rows((2, _R, 16), 1),
    )(a0, a1, a2, g0, dis, W1, b1, W2)


def _tcB2(a0, a1, g1, dis, b2, W3):
    n = dis.shape[0]

    def body(a0_ref, a1_ref, g_ref, d_ref, b2_ref, w3_ref, g2_ref):
        arefs = (a0_ref, a1_ref)
        cols = [arefs[c][0] + arefs[c][1] + g_ref[c] for c in range(2)]
        u = jnp.concatenate(cols, axis=1)
        d = d_ref[...]
        h2 = jnp.tanh(d * u + b2_ref[...])
        m3 = jnp.dot(h2, w3_ref[...], preferred_element_type=jnp.float32)
        g2_ref[0] = d * m3

    return _tc_call(
        body, n,
        jax.ShapeDtypeStruct((1, n, 16), jnp.float32),
        _agg_specs(n, 2) + [_rows((2, _R, 16), 1), _rows((_R, 1), 0),
                            _full((1, 32)), _full((32, 16))],
        _rows((1, _R, 16), 1),
    )(a0, a1, g1, dis, b2, W3)


def _tcB3(a0, g2, dis, b3, W4):
    n = dis.shape[0]

    def body(a0_ref, g_ref, d_ref, b3_ref, w4_ref, g3_ref):
        u = a0_ref[0] + a0_ref[1] + g_ref[0]
        d = d_ref[...]
        h3 = jnp.tanh(d * u + b3_ref[...])
        m4 = jnp.dot(h3, w4_ref[...], preferred_element_type=jnp.float32)
        g3 = d * m4
        g3_ref[0] = jnp.concatenate(
            [g3, jnp.zeros((_R, 8), jnp.float32)], axis=1)

    return _tc_call(
        body, n,
        jax.ShapeDtypeStruct((1, n, 16), jnp.float32),
        _agg_specs(n, 1) + [_rows((1, _R, 16), 1), _rows((_R, 1), 0),
                            _full((1, 16)), _full((16, 8))],
        _rows((1, _R, 16), 1),
    )(a0, g2, dis, b3, W4)


def _tcC(a0, g3, dis, b4, Wc, bc):
    n = dis.shape[0]

    def body(a0_ref, g_ref, d_ref, b4_ref, wc_ref, bc_ref, out_ref, h4_ref):
        u = (a0_ref[0] + a0_ref[1] + g_ref[0])[:, :8]
        d = d_ref[...]
        h4 = jnp.tanh(d * u + b4_ref[...])
        h4_ref[...] = h4
        out_ref[...] = (
            jnp.dot(h4, wc_ref[...], preferred_element_type=jnp.float32)
            + bc_ref[...])

    return _tc_call(
        body, n,
        (jax.ShapeDtypeStruct((n, 2), jnp.float32),
         jax.ShapeDtypeStruct((n, 8), jnp.float32)),
        _agg_specs(n, 1) + [_rows((1, _R, 16), 1), _rows((_R, 1), 0),
                            _full((1, 8)), _full((8, 2)), _full((1, 2))],
        (_rows((_R, 2), 0), _rows((_R, 8), 0)),
    )(a0, g3, dis, b4, Wc, bc)


def kernel(x, edge_index, W1, b1, W2, b2, W3, b3, W4, b4, Wc, bc):
    n = x.shape[0]
    e = edge_index.shape[1]

    deg_k = _make_deg_kernel(n, e)
    agg_ks = [_make_agg_kernel(n, e, c) for c in range(3)]

    deg2 = deg_k(edge_index).reshape(NC, n, 1)
    dis, g0 = _tcA(deg2, x)

    a00 = agg_ks[0](edge_index, g0)
    a01 = agg_ks[1](edge_index, g0)
    a02 = agg_ks[2](edge_index, g0)
    g1 = _tcB1(a00, a01, a02, g0, dis, W1, b1.reshape(1, -1), W2)

    a10 = agg_ks[0](edge_index, g1)
    a11 = agg_ks[1](edge_index, g1)
    g2 = _tcB2(a10, a11, g1, dis, b2.reshape(1, -1), W3)

    a20 = agg_ks[0](edge_index, g2)
    g3 = _tcB3(a20, g2, dis, b3.reshape(1, -1), W4)

    a30 = agg_ks[0](edge_index, g3)
    out, h4 = _tcC(a30, g3, dis, b4.reshape(1, -1), Wc, bc.reshape(1, -1))
    return (out, h4)


# speedup vs baseline: 25.2964x; 1.0041x over previous
import functools

import jax
import jax.numpy as jnp
from jax import lax
from jax.experimental import pallas as pl
from jax.experimental.pallas import tpu as pltpu
from jax.experimental.pallas import tpu_sc as plsc

NC = 2
NS = 16
NW = NC * NS

_R = 4000


def _make_deg_kernel(n, e):
    t = 2000
    epw = e // NW
    nt = epw // t

    mesh = plsc.VectorSubcoreMesh(core_axis_name="c", subcore_axis_name="s")

    @functools.partial(
        pl.kernel,
        out_type=jax.ShapeDtypeStruct((NC, n), jnp.float32),
        mesh=mesh,
        compiler_params=pltpu.CompilerParams(use_tc_tiling_on_sc=False),
        scratch_types=[
            pltpu.VMEM((t,), jnp.int32),
            pltpu.VMEM((t,), jnp.float32),
            pltpu.VMEM_SHARED((n,), jnp.float32),
        ],
    )
    def deg_kernel(edge_hbm, out_hbm, dst_v, one_v, acc_sh):
        c = lax.axis_index("c")
        s = lax.axis_index("s")

        def fill(val, i, _):
            one_v[pl.ds(i * 16, 16)] = jnp.full((16,), val, jnp.float32)
            return 0

        @pl.when(s == 0)
        def _():
            lax.fori_loop(0, t // 16, functools.partial(fill, 0.0), 0)
            for k in range(n // t):
                pltpu.sync_copy(one_v, acc_sh.at[pl.ds(k * t, t)])

        lax.fori_loop(0, t // 16, functools.partial(fill, 1.0), 0)
        plsc.subcore_barrier()

        base = (c * NS + s) * epw

        def body(i, _):
            pltpu.sync_copy(edge_hbm.at[1, pl.ds(base + i * t, t)], dst_v)
            pltpu.sync_copy(one_v, acc_sh.at[dst_v], add=True)
            return 0

        lax.fori_loop(0, nt, body, 0)
        plsc.subcore_barrier()

        @pl.when(s == 0)
        def _():
            pltpu.sync_copy(acc_sh, out_hbm.at[c])

    return deg_kernel


def _make_agg_kernel(n, e, chunk):
    t = 200
    k_slots = 5
    tb = t * k_slots
    epw = e // NW
    nb = epw // tb
    n_pad = ((n + 8 * NS - 1) // (8 * NS)) * (8 * NS)
    rpt = n_pad // NS

    mesh = plsc.VectorSubcoreMesh(core_axis_name="c", subcore_axis_name="s")

    @functools.partial(
        pl.kernel,
        out_type=jax.ShapeDtypeStruct((NC, n_pad, 16), jnp.float32),
        mesh=mesh,
        compiler_params=pltpu.CompilerParams(use_tc_tiling_on_sc=False),
        scratch_types=[
            pltpu.VMEM((k_slots, 2, t), jnp.int32),
            pltpu.VMEM((tb, 16), jnp.float32),
            pltpu.VMEM_SHARED((n_pad, 16), jnp.float32),
            pltpu.SemaphoreType.DMA((k_slots,)),
            pltpu.SemaphoreType.DMA((k_slots,)),
            pltpu.SemaphoreType.DMA((k_slots,)),
        ],
    )
    def agg_kernel(edge_hbm, tab3_hbm, out_hbm,
                   idx_v, rows_v, acc_sh, isem, gsem, ssem):
        c = lax.axis_index("c")
        s = lax.axis_index("s")
        tab_hbm = tab3_hbm.at[chunk]

        def zfill(i, _):
            rows_v[i] = jnp.zeros((16,), jnp.float32)
            return 0

        lax.fori_loop(0, tb, zfill, 0)
        row0 = s * rpt
        nfull, rem = divmod(rpt, tb)
        for k in range(nfull):
            pltpu.sync_copy(rows_v, acc_sh.at[pl.ds(row0 + k * tb, tb)])
        if rem:
            pltpu.sync_copy(rows_v.at[pl.ds(0, rem)],
                            acc_sh.at[pl.ds(row0 + nfull * tb, rem)])
        plsc.subcore_barrier()

        base = (c * NS + s) * epw

        def body(m, _):
            off0 = base + m * tb
            ic, gc, sc = [], [], []
            for b in range(k_slots):
                ic.append(pltpu.async_copy(
                    edge_hbm.at[:, pl.ds(off0 + b * t, t)],
                    idx_v.at[b], isem.at[b]))
            for b in range(k_slots):
                ic[b].wait()
                gc.append(pltpu.async_copy(
                    tab_hbm.at[idx_v.at[b, 0]],
                    rows_v.at[pl.ds(b * t, t)], gsem.at[b]))
            for b in range(k_slots):
                gc[b].wait()
                sc.append(pltpu.async_copy(
                    rows_v.at[pl.ds(b * t, t)],
                    acc_sh.at[idx_v.at[b, 1]], ssem.at[b], add=True))
            for b in range(k_slots):
                sc[b].wait()
            return 0

        lax.fori_loop(0, nb, body, 0)
        plsc.subcore_barrier()
        pltpu.sync_copy(acc_sh.at[pl.ds(row0, rpt)],
                        out_hbm.at[c, pl.ds(row0, rpt)])

    return agg_kernel


def _tc_call(body, n, out_shapes, in_specs, out_specs):
    grid = (n // _R,)
    return pl.pallas_call(
        body,
        grid=grid,
        out_shape=out_shapes,
        in_specs=in_specs,
        out_specs=out_specs,
    )


def _full(shape):
    return pl.BlockSpec(shape, lambda i: tuple(0 for _ in shape))


def _rows(shape, axis):
    def imap(i, axis=axis):
        return tuple(i if a == axis else 0 for a in range(len(shape)))
    return pl.BlockSpec(shape, imap)


def _tcA(deg2, x):
    n = x.shape[0]

    def body(deg_ref, x_ref, dis_ref, g0_ref):
        deg = deg_ref[0] + deg_ref[1] + 1.0
        dis = lax.rsqrt(deg)
        dis_ref[...] = dis
        xp = jnp.concatenate(
            [x_ref[...], jnp.zeros((_R, 14), jnp.float32)], axis=1)
        g = dis * xp
        for c in range(3):
            g0_ref[c] = g[:, c * 16:(c + 1) * 16]

    return _tc_call(
        body, n,
        (jax.ShapeDtypeStruct((n, 1), jnp.float32),
         jax.ShapeDtypeStruct((3, n, 16), jnp.float32)),
        [_rows((2, _R, 1), 1), _rows((_R, 34), 0)],
        (_rows((_R, 1), 0), _rows((3, _R, 16), 1)),
    )(deg2, x)


def _agg_specs(n_arr, count):
    return [_rows((2, _R, 16), 1) for _ in range(count)]


def _tcB1(a0, a1, a2, g0, dis, W1, b1, W2):
    n = dis.shape[0]

    def body(a0_ref, a1_ref, a2_ref, g_ref, d_ref,
             w1_ref, b1_ref, w2_ref, g1_ref):
        arefs = (a0_ref, a1_ref, a2_ref)
        cols = [arefs[c][0] + arefs[c][1] + g_ref[c] for c in range(3)]
        u = jnp.concatenate(cols, axis=1)
        d = d_ref[...]
        pre = (d * u)[:, :34]
        h1 = jnp.tanh(
            jnp.dot(pre, w1_ref[...], preferred_element_type=jnp.float32)
            + b1_ref[...])
        m2 = jnp.dot(h1, w2_ref[...], preferred_element_type=jnp.float32)
        g1 = d * m2
        g1_ref[0] = g1[:, :16]
        g1_ref[1] = g1[:, 16:]

    return _tc_call(
        body, n,
        jax.ShapeDtypeStruct((2, n, 16), jnp.float32),
        _agg_specs(n, 3) + [_rows((3, _R, 16), 1), _rows((_R, 1), 0),
                            _full((34, 64)), _full((1, 64)), _full((64, 32))],
        _rows((2, _R, 16), 1),
    )(a0, a1, a2, g0, dis, W1, b1, W2)


def _tcB2(a0, a1, g1, dis, b2, W3):
    n = dis.shape[0]

    def body(a0_ref, a1_ref, g_ref, d_ref, b2_ref, w3_ref, g2_ref):
        arefs = (a0_ref, a1_ref)
        cols = [arefs[c][0] + arefs[c][1] + g_ref[c] for c in range(2)]
        u = jnp.concatenate(cols, axis=1)
        d = d_ref[...]
        h2 = jnp.tanh(d * u + b2_ref[...])
        m3 = jnp.dot(h2, w3_ref[...], preferred_element_type=jnp.float32)
        g2_ref[0] = d * m3

    return _tc_call(
        body, n,
        jax.ShapeDtypeStruct((1, n, 16), jnp.float32),
        _agg_specs(n, 2) + [_rows((2, _R, 16), 1), _rows((_R, 1), 0),
                            _full((1, 32)), _full((32, 16))],
        _rows((1, _R, 16), 1),
    )(a0, a1, g1, dis, b2, W3)


def _tcB3(a0, g2, dis, b3, W4):
    n = dis.shape[0]

    def body(a0_ref, g_ref, d_ref, b3_ref, w4_ref, g3_ref):
        u = a0_ref[0] + a0_ref[1] + g_ref[0]
        d = d_ref[...]
        h3 = jnp.tanh(d * u + b3_ref[...])
        m4 = jnp.dot(h3, w4_ref[...], preferred_element_type=jnp.float32)
        g3 = d * m4
        g3_ref[0] = jnp.concatenate(
            [g3, jnp.zeros((_R, 8), jnp.float32)], axis=1)

    return _tc_call(
        body, n,
        jax.ShapeDtypeStruct((1, n, 16), jnp.float32),
        _agg_specs(n, 1) + [_rows((1, _R, 16), 1), _rows((_R, 1), 0),
                            _full((1, 16)), _full((16, 8))],
        _rows((1, _R, 16), 1),
    )(a0, g2, dis, b3, W4)


def _tcC(a0, g3, dis, b4, Wc, bc):
    n = dis.shape[0]

    def body(a0_ref, g_ref, d_ref, b4_ref, wc_ref, bc_ref, out_ref, h4_ref):
        u = (a0_ref[0] + a0_ref[1] + g_ref[0])[:, :8]
        d = d_ref[...]
        h4 = jnp.tanh(d * u + b4_ref[...])
        h4_ref[...] = h4
        out_ref[...] = (
            jnp.dot(h4, wc_ref[...], preferred_element_type=jnp.float32)
            + bc_ref[...])

    return _tc_call(
        body, n,
        (jax.ShapeDtypeStruct((n, 2), jnp.float32),
         jax.ShapeDtypeStruct((n, 8), jnp.float32)),
        _agg_specs(n, 1) + [_rows((1, _R, 16), 1), _rows((_R, 1), 0),
                            _full((1, 8)), _full((8, 2)), _full((1, 2))],
        (_rows((_R, 2), 0), _rows((_R, 8), 0)),
    )(a0, g3, dis, b4, Wc, bc)


def kernel(x, edge_index, W1, b1, W2, b2, W3, b3, W4, b4, Wc, bc):
    n = x.shape[0]
    e = edge_index.shape[1]

    deg_k = _make_deg_kernel(n, e)
    agg_ks = [_make_agg_kernel(n, e, c) for c in range(3)]

    deg2 = deg_k(edge_index).reshape(NC, n, 1)
    dis, g0 = _tcA(deg2, x)

    a00 = agg_ks[0](edge_index, g0)
    a01 = agg_ks[1](edge_index, g0)
    a02 = agg_ks[2](edge_index, g0)
    g1 = _tcB1(a00, a01, a02, g0, dis, W1, b1.reshape(1, -1), W2)

    a10 = agg_ks[0](edge_index, g1)
    a11 = agg_ks[1](edge_index, g1)
    g2 = _tcB2(a10, a11, g1, dis, b2.reshape(1, -1), W3)

    a20 = agg_ks[0](edge_index, g2)
    g3 = _tcB3(a20, g2, dis, b3.reshape(1, -1), W4)

    a30 = agg_ks[0](edge_index, g3)
    out, h4 = _tcC(a30, g3, dis, b4.reshape(1, -1), Wc, bc.reshape(1, -1))
    return (out, h4)


# speedup vs baseline: 26.5522x; 1.0496x over previous
import functools

import jax
import jax.numpy as jnp
from jax import lax
from jax.experimental import pallas as pl
from jax.experimental.pallas import tpu as pltpu
from jax.experimental.pallas import tpu_sc as plsc

NC = 2
NS = 16
NW = NC * NS

_R = 4000


def _make_deg_kernel(n, e):
    t = 2000
    epw = e // NW
    nt = epw // t

    mesh = plsc.VectorSubcoreMesh(core_axis_name="c", subcore_axis_name="s")

    @functools.partial(
        pl.kernel,
        out_type=jax.ShapeDtypeStruct((NC, n), jnp.float32),
        mesh=mesh,
        compiler_params=pltpu.CompilerParams(use_tc_tiling_on_sc=False),
        scratch_types=[
            pltpu.VMEM((t,), jnp.int32),
            pltpu.VMEM((t,), jnp.float32),
            pltpu.VMEM_SHARED((n,), jnp.float32),
        ],
    )
    def deg_kernel(edge_hbm, out_hbm, dst_v, one_v, acc_sh):
        c = lax.axis_index("c")
        s = lax.axis_index("s")

        def fill(val, i, _):
            one_v[pl.ds(i * 16, 16)] = jnp.full((16,), val, jnp.float32)
            return 0

        @pl.when(s == 0)
        def _():
            lax.fori_loop(0, t // 16, functools.partial(fill, 0.0), 0)
            for k in range(n // t):
                pltpu.sync_copy(one_v, acc_sh.at[pl.ds(k * t, t)])

        lax.fori_loop(0, t // 16, functools.partial(fill, 1.0), 0)
        plsc.subcore_barrier()

        base = (c * NS + s) * epw

        def body(i, _):
            pltpu.sync_copy(edge_hbm.at[1, pl.ds(base + i * t, t)], dst_v)
            pltpu.sync_copy(one_v, acc_sh.at[dst_v], add=True)
            return 0

        lax.fori_loop(0, nt, body, 0)
        plsc.subcore_barrier()

        @pl.when(s == 0)
        def _():
            pltpu.sync_copy(acc_sh, out_hbm.at[c])

    return deg_kernel


def _make_agg_kernel(n, e, chunk):
    t = 200
    k_slots = 5
    tb = t * k_slots
    epw = e // NW
    nb = epw // tb
    n_pad = ((n + 8 * NS - 1) // (8 * NS)) * (8 * NS)
    rpt = n_pad // NS

    mesh = plsc.VectorSubcoreMesh(core_axis_name="c", subcore_axis_name="s")

    @functools.partial(
        pl.kernel,
        out_type=jax.ShapeDtypeStruct((NC, n_pad, 16), jnp.float32),
        mesh=mesh,
        compiler_params=pltpu.CompilerParams(use_tc_tiling_on_sc=False),
        scratch_types=[
            pltpu.VMEM((k_slots, 2, t), jnp.int32),
            pltpu.VMEM((tb, 16), jnp.float32),
            pltpu.VMEM_SHARED((n_pad, 16), jnp.float32),
            pltpu.SemaphoreType.DMA((k_slots,)),
            pltpu.SemaphoreType.DMA((k_slots,)),
            pltpu.SemaphoreType.DMA((k_slots,)),
        ],
    )
    def agg_kernel(edge_hbm, tab3_hbm, out_hbm,
                   idx_v, rows_v, acc_sh, isem, gsem, ssem):
        c = lax.axis_index("c")
        s = lax.axis_index("s")
        tab_hbm = tab3_hbm.at[chunk]

        def zfill(i, _):
            rows_v[i] = jnp.zeros((16,), jnp.float32)
            return 0

        lax.fori_loop(0, tb, zfill, 0)
        row0 = s * rpt
        nfull, rem = divmod(rpt, tb)
        for k in range(nfull):
            pltpu.sync_copy(rows_v, acc_sh.at[pl.ds(row0 + k * tb, tb)])
        if rem:
            pltpu.sync_copy(rows_v.at[pl.ds(0, rem)],
                            acc_sh.at[pl.ds(row0 + nfull * tb, rem)])
        plsc.subcore_barrier()

        base = (c * NS + s) * epw

        def _drain_scatter(b):
            pltpu.make_async_copy(
                rows_v.at[pl.ds(b * t, t)],
                acc_sh.at[pl.ds(0, t)], ssem.at[b]).wait()

        def body(m, _):
            off0 = base + m * tb
            ic, gc = [], []
            for b in range(k_slots):
                @pl.when(m > 0)
                def _(b=b):
                    _drain_scatter(b)

                ic.append(pltpu.async_copy(
                    edge_hbm.at[:, pl.ds(off0 + b * t, t)],
                    idx_v.at[b], isem.at[b]))
            for b in range(k_slots):
                ic[b].wait()
                gc.append(pltpu.async_copy(
                    tab_hbm.at[idx_v.at[b, 0]],
                    rows_v.at[pl.ds(b * t, t)], gsem.at[b]))
            for b in range(k_slots):
                gc[b].wait()
                pltpu.async_copy(
                    rows_v.at[pl.ds(b * t, t)],
                    acc_sh.at[idx_v.at[b, 1]], ssem.at[b], add=True)
            return 0

        lax.fori_loop(0, nb, body, 0)
        for b in range(k_slots):
            _drain_scatter(b)
        plsc.subcore_barrier()
        pltpu.sync_copy(acc_sh.at[pl.ds(row0, rpt)],
                        out_hbm.at[c, pl.ds(row0, rpt)])

    return agg_kernel


def _tc_call(body, n, out_shapes, in_specs, out_specs):
    grid = (n // _R,)
    return pl.pallas_call(
        body,
        grid=grid,
        out_shape=out_shapes,
        in_specs=in_specs,
        out_specs=out_specs,
    )


def _full(shape):
    return pl.BlockSpec(shape, lambda i: tuple(0 for _ in shape))


def _rows(shape, axis):
    def imap(i, axis=axis):
        return tuple(i if a == axis else 0 for a in range(len(shape)))
    return pl.BlockSpec(shape, imap)


def _tcA(deg2, x):
    n = x.shape[0]

    def body(deg_ref, x_ref, dis_ref, g0_ref):
        deg = deg_ref[0] + deg_ref[1] + 1.0
        dis = lax.rsqrt(deg)
        dis_ref[...] = dis
        xp = jnp.concatenate(
            [x_ref[...], jnp.zeros((_R, 14), jnp.float32)], axis=1)
        g = dis * xp
        for c in range(3):
            g0_ref[c] = g[:, c * 16:(c + 1) * 16]

    return _tc_call(
        body, n,
        (jax.ShapeDtypeStruct((n, 1), jnp.float32),
         jax.ShapeDtypeStruct((3, n, 16), jnp.float32)),
        [_rows((2, _R, 1), 1), _rows((_R, 34), 0)],
        (_rows((_R, 1), 0), _rows((3, _R, 16), 1)),
    )(deg2, x)


def _agg_specs(n_arr, count):
    return [_rows((2, _R, 16), 1) for _ in range(count)]


def _tcB1(a0, a1, a2, g0, dis, W1, b1, W2):
    n = dis.shape[0]

    def body(a0_ref, a1_ref, a2_ref, g_ref, d_ref,
             w1_ref, b1_ref, w2_ref, g1_ref):
        arefs = (a0_ref, a1_ref, a2_ref)
        cols = [arefs[c][0] + arefs[c][1] + g_ref[c] for c in range(3)]
        u = jnp.concatenate(cols, axis=1)
        d = d_ref[...]
        pre = (d * u)[:, :34]
        h1 = jnp.tanh(
            jnp.dot(pre, w1_ref[...], preferred_element_type=jnp.float32)
            + b1_ref[...])
        m2 = jnp.dot(h1, w2_ref[...], preferred_element_type=jnp.float32)
        g1 = d * m2
        g1_ref[0] = g1[:, :16]
        g1_ref[1] = g1[:, 16:]

    return _tc_call(
        body, n,
        jax.ShapeDtypeStruct((2, n, 16), jnp.float32),
        _agg_specs(n, 3) + [_rows((3, _R, 16), 1), _rows((_R, 1), 0),
                            _full((34, 64)), _full((1, 64)), _full((64, 32))],
        _rows((2, _R, 16), 1),
    )(a0, a1, a2, g0, dis, W1, b1, W2)


def _tcB2(a0, a1, g1, dis, b2, W3):
    n = dis.shape[0]

    def body(a0_ref, a1_ref, g_ref, d_ref, b2_ref, w3_ref, g2_ref):
        arefs = (a0_ref, a1_ref)
        cols = [arefs[c][0] + arefs[c][1] + g_ref[c] for c in range(2)]
        u = jnp.concatenate(cols, axis=1)
        d = d_ref[...]
        h2 = jnp.tanh(d * u + b2_ref[...])
        m3 = jnp.dot(h2, w3_ref[...], preferred_element_type=jnp.float32)
        g2_ref[0] = d * m3

    return _tc_call(
        body, n,
        jax.ShapeDtypeStruct((1, n, 16), jnp.float32),
        _agg_specs(n, 2) + [_rows((2, _R, 16), 1), _rows((_R, 1), 0),
                            _full((1, 32)), _full((32, 16))],
        _rows((1, _R, 16), 1),
    )(a0, a1, g1, dis, b2, W3)


def _tcB3(a0, g2, dis, b3, W4):
    n = dis.shape[0]

    def body(a0_ref, g_ref, d_ref, b3_ref, w4_ref, g3_ref):
        u = a0_ref[0] + a0_ref[1] + g_ref[0]
        d = d_ref[...]
        h3 = jnp.tanh(d * u + b3_ref[...])
        m4 = jnp.dot(h3, w4_ref[...], preferred_element_type=jnp.float32)
        g3 = d * m4
        g3_ref[0] = jnp.concatenate(
            [g3, jnp.zeros((_R, 8), jnp.float32)], axis=1)

    return _tc_call(
        body, n,
        jax.ShapeDtypeStruct((1, n, 16), jnp.float32),
        _agg_specs(n, 1) + [_rows((1, _R, 16), 1), _rows((_R, 1), 0),
                            _full((1, 16)), _full((16, 8))],
        _rows((1, _R, 16), 1),
    )(a0, g2, dis, b3, W4)


def _tcC(a0, g3, dis, b4, Wc, bc):
    n = dis.shape[0]

    def body(a0_ref, g_ref, d_ref, b4_ref, wc_ref, bc_ref, out_ref, h4_ref):
        u = (a0_ref[0] + a0_ref[1] + g_ref[0])[:, :8]
        d = d_ref[...]
        h4 = jnp.tanh(d * u + b4_ref[...])
        h4_ref[...] = h4
        out_ref[...] = (
            jnp.dot(h4, wc_ref[...], preferred_element_type=jnp.float32)
            + bc_ref[...])

    return _tc_call(
        body, n,
        (jax.ShapeDtypeStruct((n, 2), jnp.float32),
         jax.ShapeDtypeStruct((n, 8), jnp.float32)),
        _agg_specs(n, 1) + [_rows((1, _R, 16), 1), _rows((_R, 1), 0),
                            _full((1, 8)), _full((8, 2)), _full((1, 2))],
        (_rows((_R, 2), 0), _rows((_R, 8), 0)),
    )(a0, g3, dis, b4, Wc, bc)


def kernel(x, edge_index, W1, b1, W2, b2, W3, b3, W4, b4, Wc, bc):
    n = x.shape[0]
    e = edge_index.shape[1]

    deg_k = _make_deg_kernel(n, e)
    agg_ks = [_make_agg_kernel(n, e, c) for c in range(3)]

    deg2 = deg_k(edge_index).reshape(NC, n, 1)
    dis, g0 = _tcA(deg2, x)

    a00 = agg_ks[0](edge_index, g0)
    a01 = agg_ks[1](edge_index, g0)
    a02 = agg_ks[2](edge_index, g0)
    g1 = _tcB1(a00, a01, a02, g0, dis, W1, b1.reshape(1, -1), W2)

    a10 = agg_ks[0](edge_index, g1)
    a11 = agg_ks[1](edge_index, g1)
    g2 = _tcB2(a10, a11, g1, dis, b2.reshape(1, -1), W3)

    a20 = agg_ks[0](edge_index, g2)
    g3 = _tcB3(a20, g2, dis, b3.reshape(1, -1), W4)

    a30 = agg_ks[0](edge_index, g3)
    out, h4 = _tcC(a30, g3, dis, b4.reshape(1, -1), Wc, bc.reshape(1, -1))
    return (out, h4)
